# baseline (speedup 1.0000x reference)
import functools

import jax
import jax.numpy as jnp
from jax import lax
from jax.experimental import pallas as pl
from jax.experimental.pallas import tpu as pltpu
from jax.experimental.pallas import tpu_sc as plsc

_H = 256
_E = 8
_T = 8192
_TB = 1024
_HA = 384
_NW = 32
_CHUNK = _T // _NW


def _router_kernel(x_ref, wg_ref, xg_ref, idx_ref):
    xb = x_ref[...]
    logits = jnp.dot(xb, wg_ref[...], preferred_element_type=jnp.float32)
    m = jnp.max(logits, axis=-1, keepdims=True)
    ex = jnp.exp(logits - m)
    s = jnp.sum(ex, axis=-1, keepdims=True)
    g = jnp.max(ex, axis=-1, keepdims=True) / s
    idx = jnp.argmax(logits, axis=-1)
    xg_ref[...] = jnp.concatenate(
        [g * xb, g, jnp.zeros((_TB, _HA - _H - 1), jnp.float32)], axis=1)
    idx_ref[...] = idx.astype(jnp.int32).reshape(_TB // 128, 128)


def _hist_kernel(idx_hbm, h_hbm, idx_v, hv_v):
    wid = lax.axis_index("s") * 2 + lax.axis_index("c")
    pltpu.sync_copy(idx_hbm.at[pl.ds(2 * wid, 2)], idx_v)
    li = lax.iota(jnp.int32, 16)
    hv = jnp.zeros((16,), jnp.int32)
    for r in range(2):
        for k in range(8):
            vec = idx_v[r, pl.ds(16 * k, 16)]
            for lane in range(16):
                idx_b = vec.at[jnp.full((16,), lane, jnp.int32)].get(
                    mode="promise_in_bounds")
                hv = hv + jnp.where(li == idx_b, 1, 0)
    hv_v[...] = hv
    pltpu.sync_copy(hv_v, h_hbm.at[wid])


def _dispatch_kernel(idx_hbm, xg_hbm, h_hbm,
                     xs_hbm, dest_hbm, offs_hbm, lo_hbm, hi_hbm,
                     h_v, idx_v, dest_v, xrv, meta_v, sem):
    wid = lax.axis_index("s") * 2 + lax.axis_index("c")
    li = lax.iota(jnp.int32, 16)

    pltpu.sync_copy(h_hbm, h_v)
    totals = jnp.zeros((16,), jnp.int32)
    run_pre = jnp.zeros((16,), jnp.int32)
    for w in range(_NW):
        hrow = h_v[w, :]
        totals = totals + hrow
        run_pre = run_pre + jnp.where(w < wid, hrow, 0)
    base = jnp.zeros((16,), jnp.int32)
    for e in range(_E):
        tot_e = totals.at[jnp.full((16,), e, jnp.int32)].get(
            mode="promise_in_bounds")
        base = base + jnp.where(li > e, tot_e, 0)
    run = base + run_pre

    pltpu.sync_copy(idx_hbm.at[pl.ds(2 * wid, 2)], idx_v)
    for r in range(2):
        for k in range(8):
            vec = idx_v[r, pl.ds(16 * k, 16)]
            dvec = jnp.zeros((16,), jnp.int32)
            for lane in range(16):
                idx_b = vec.at[jnp.full((16,), lane, jnp.int32)].get(
                    mode="promise_in_bounds")
                d_b = run.at[idx_b].get(mode="promise_in_bounds")
                dvec = jnp.where(li == lane, d_b, dvec)
                run = run + jnp.where(li == idx_b, 1, 0)
            dest_v[r, pl.ds(16 * k, 16)] = dvec

    pltpu.sync_copy(dest_v, dest_hbm.at[pl.ds(2 * wid, 2)])
    pltpu.sync_copy(xg_hbm.at[pl.ds(_CHUNK * wid, _CHUNK)], xrv)
    for j in range(2):
        pltpu.async_copy(xrv.at[pl.ds(j * 128, 128)],
                         xs_hbm.at[dest_v.at[j]], sem).wait()

    @pl.when(wid == 0)
    def _():
        nblk = _T // _TB
        row0 = li * _TB
        lo = jnp.zeros((16,), jnp.int32)
        hi = jnp.zeros((16,), jnp.int32)
        for e in range(_E):
            base_e = base.at[jnp.full((16,), e, jnp.int32)].get(
                mode="promise_in_bounds")
            lo = lo + jnp.where(base_e <= row0, 1, 0)
            hi = hi + jnp.where(base_e < row0 + _TB, 1, 0)
        meta_v[0, :] = base
        meta_v[1, :] = lo - 1
        meta_v[2, :] = hi - 1
        pltpu.sync_copy(meta_v.at[0], offs_hbm)
        pltpu.sync_copy(meta_v.at[1], lo_hbm)
        pltpu.sync_copy(meta_v.at[2], hi_hbm)
        del nblk


def _group_mm_kernel(offs_ref, lo_ref, hi_ref, xs_ref, we_ref, be_ref, ys_ref):
    b = pl.program_id(0)
    xb = xs_ref[...]
    xb16 = xb[:, :_H].astype(jnp.bfloat16)
    gsb = xb[:, _H:_H + 1]
    rowid = _TB * b + lax.broadcasted_iota(jnp.int32, (_TB, 1), 0)
    lo = lo_ref[b]
    hi = hi_ref[b]
    ys_ref[...] = jnp.zeros((_TB, _H), jnp.float32)
    for e in range(_E):
        @pl.when(jnp.logical_and(lo <= e, e <= hi))
        def _():
            y = jnp.dot(xb16, we_ref[e], preferred_element_type=jnp.float32)
            seg = jnp.logical_and(rowid >= offs_ref[e], rowid < offs_ref[e + 1])
            ys_ref[...] = jnp.where(seg, y + gsb * be_ref[e][None, :],
                                    ys_ref[...])


def _combine_kernel(ys_hbm, dest_hbm, out_hbm, dest_v, yrv, sem):
    wid = lax.axis_index("s") * 2 + lax.axis_index("c")
    pltpu.sync_copy(dest_hbm.at[pl.ds(2 * wid, 2)], dest_v)
    for j in range(2):
        pltpu.async_copy(ys_hbm.at[dest_v.at[j]],
                         yrv.at[pl.ds(j * 128, 128)], sem).wait()
    pltpu.sync_copy(yrv, out_hbm.at[pl.ds(_CHUNK * wid, _CHUNK)])


_SC_MESH = plsc.VectorSubcoreMesh(core_axis_name="c", subcore_axis_name="s")


@jax.jit
def kernel(x, Wg, We, be):
    B, S, H = x.shape
    xt = x.reshape(_T, H)
    we16 = We.astype(jnp.bfloat16)

    xg, idxu = pl.pallas_call(
        _router_kernel,
        grid=(_T // _TB,),
        in_specs=[
            pl.BlockSpec((_TB, H), lambda i: (i, 0)),
            pl.BlockSpec((H, _E), lambda i: (0, 0)),
        ],
        out_specs=(
            pl.BlockSpec((_TB, _HA), lambda i: (i, 0)),
            pl.BlockSpec((_TB // 128, 128), lambda i: (i, 0)),
        ),
        out_shape=(
            jax.ShapeDtypeStruct((_T, _HA), jnp.float32),
            jax.ShapeDtypeStruct((_T // 128, 128), jnp.int32),
        ),
    )(xt, Wg)

    hist = pl.kernel(
        _hist_kernel,
        mesh=_SC_MESH,
        out_type=jax.ShapeDtypeStruct((_NW, 16), jnp.int32),
        scratch_types=[
            pltpu.VMEM((2, 128), jnp.int32),
            pltpu.VMEM((16,), jnp.int32),
        ],
    )
    h = hist(idxu)

    dispatch = pl.kernel(
        _dispatch_kernel,
        mesh=_SC_MESH,
        out_type=(
            jax.ShapeDtypeStruct((_T, _HA), jnp.float32),
            jax.ShapeDtypeStruct((_T // 128, 128), jnp.int32),
            jax.ShapeDtypeStruct((16,), jnp.int32),
            jax.ShapeDtypeStruct((16,), jnp.int32),
            jax.ShapeDtypeStruct((16,), jnp.int32),
        ),
        scratch_types=[
            pltpu.VMEM((_NW, 16), jnp.int32),
            pltpu.VMEM((2, 128), jnp.int32),
            pltpu.VMEM((2, 128), jnp.int32),
            pltpu.VMEM((_CHUNK, _HA), jnp.float32),
            pltpu.VMEM((3, 16), jnp.int32),
            pltpu.SemaphoreType.DMA,
        ],
    )
    xs, dest, offs, lo, hi = dispatch(idxu, xg, h)

    ys = pl.pallas_call(
        _group_mm_kernel,
        grid=(_T // _TB,),
        in_specs=[
            pl.BlockSpec(memory_space=pltpu.SMEM),
            pl.BlockSpec(memory_space=pltpu.SMEM),
            pl.BlockSpec(memory_space=pltpu.SMEM),
            pl.BlockSpec((_TB, _HA), lambda i: (i, 0)),
            pl.BlockSpec((_E, H, H), lambda i: (0, 0, 0)),
            pl.BlockSpec((_E, H), lambda i: (0, 0)),
        ],
        out_specs=pl.BlockSpec((_TB, H), lambda i: (i, 0)),
        out_shape=jax.ShapeDtypeStruct((_T, H), jnp.float32),
    )(offs, lo, hi, xs, we16, be)

    combine = pl.kernel(
        _combine_kernel,
        mesh=_SC_MESH,
        out_type=jax.ShapeDtypeStruct((_T, H), jnp.float32),
        scratch_types=[
            pltpu.VMEM((2, 128), jnp.int32),
            pltpu.VMEM((_CHUNK, H), jnp.float32),
            pltpu.SemaphoreType.DMA,
        ],
    )
    out = combine(ys, dest)
    return out.reshape(B, S, H)


# speedup vs baseline: 3.8052x; 3.8052x over previous
import jax
import jax.numpy as jnp
from jax.experimental import pallas as pl

_HIDDEN = 256
_NUM_EXPERTS = 8
_TB = 4096


def _moe_block_kernel(x_ref, wg_ref, we_ref, be_ref, out_ref):
    xb = x_ref[...]
    logits = jnp.dot(xb, wg_ref[...], preferred_element_type=jnp.float32)
    m = jnp.max(logits, axis=-1, keepdims=True)
    e = jnp.exp(logits - m)
    s = jnp.sum(e, axis=-1, keepdims=True)
    gates = e / s
    idx = jnp.argmax(logits, axis=-1)
    gate_val = jnp.max(gates, axis=-1)

    acc = jnp.zeros((_TB, _HIDDEN), dtype=jnp.float32)
    xb16 = xb.astype(jnp.bfloat16)
    for ex in range(_NUM_EXPERTS):
        y = jnp.dot(xb16, we_ref[ex].astype(jnp.bfloat16),
                    preferred_element_type=jnp.float32)
        mask = (idx == ex)[:, None]
        acc = acc + jnp.where(mask, y, 0.0)
    oh = (idx[:, None] ==
          jax.lax.broadcasted_iota(jnp.int32, (_TB, _NUM_EXPERTS), 1))
    bias = jnp.dot(oh.astype(jnp.float32), be_ref[...],
                   preferred_element_type=jnp.float32)
    out_ref[...] = gate_val[:, None] * (acc + bias)


@jax.jit
def kernel(x, Wg, We, be):
    B, S, H = x.shape
    T = B * S
    xt = x.reshape(T, H)
    grid = (T // _TB,)
    out = pl.pallas_call(
        _moe_block_kernel,
        grid=grid,
        in_specs=[
            pl.BlockSpec((_TB, H), lambda i: (i, 0)),
            pl.BlockSpec((H, _NUM_EXPERTS), lambda i: (0, 0)),
            pl.BlockSpec((_NUM_EXPERTS, H, H), lambda i: (0, 0, 0)),
            pl.BlockSpec((_NUM_EXPERTS, H), lambda i: (0, 0)),
        ],
        out_specs=pl.BlockSpec((_TB, H), lambda i: (i, 0)),
        out_shape=jax.ShapeDtypeStruct((T, H), jnp.float32),
    )(xt, Wg, We, be)
    return out.reshape(B, S, H)


# speedup vs baseline: 3.9657x; 1.0422x over previous
import jax
import jax.numpy as jnp
from jax.experimental import pallas as pl

_HIDDEN = 256
_NUM_EXPERTS = 8
_TB = 2048


def _moe_block_kernel(x_ref, wg_ref, we_ref, be_ref, out_ref):
    xb = x_ref[...]
    logits = jnp.dot(xb, wg_ref[...], preferred_element_type=jnp.float32)
    m = jnp.max(logits, axis=-1, keepdims=True)
    e = jnp.exp(logits - m)
    s = jnp.sum(e, axis=-1, keepdims=True)
    gates = e / s
    idx = jnp.argmax(logits, axis=-1)
    gate_val = jnp.max(gates, axis=-1)

    acc = jnp.zeros((_TB, _HIDDEN), dtype=jnp.float32)
    xb16 = xb.astype(jnp.bfloat16)
    for ex in range(_NUM_EXPERTS):
        y = jnp.dot(xb16, we_ref[ex].astype(jnp.bfloat16),
                    preferred_element_type=jnp.float32)
        mask = (idx == ex)[:, None]
        acc = acc + jnp.where(mask, y, 0.0)
    oh = (idx[:, None] ==
          jax.lax.broadcasted_iota(jnp.int32, (_TB, _NUM_EXPERTS), 1))
    bias = jnp.dot(oh.astype(jnp.float32), be_ref[...],
                   preferred_element_type=jnp.float32)
    out_ref[...] = gate_val[:, None] * (acc + bias)


@jax.jit
def kernel(x, Wg, We, be):
    B, S, H = x.shape
    T = B * S
    xt = x.reshape(T, H)
    grid = (T // _TB,)
    out = pl.pallas_call(
        _moe_block_kernel,
        grid=grid,
        in_specs=[
            pl.BlockSpec((_TB, H), lambda i: (i, 0)),
            pl.BlockSpec((H, _NUM_EXPERTS), lambda i: (0, 0)),
            pl.BlockSpec((_NUM_EXPERTS, H, H), lambda i: (0, 0, 0)),
            pl.BlockSpec((_NUM_EXPERTS, H), lambda i: (0, 0)),
        ],
        out_specs=pl.BlockSpec((_TB, H), lambda i: (i, 0)),
        out_shape=jax.ShapeDtypeStruct((T, H), jnp.float32),
    )(xt, Wg, We, be)
    return out.reshape(B, S, H)
